# baseline (speedup 1.0000x reference)
import jax
import jax.numpy as jnp
from jax.experimental import pallas as pl


def _gcn_fused_kernel(xef_ref, xev_ref, xnc_ref, px_ref, pv_ref, i8_ref,
                      w8_ref, ne2_ref, wco2_ref, we2_ref, ws2_ref, wd2_ref,
                      bm2_ref, wn2_ref, bn2_ref, wce2_ref, wcs2_ref,
                      wcd2_ref, out_ref):
    NN = xef_ref.shape[1]
    N = out_ref.shape[1]
    H2 = ne2_ref.shape[1]
    xef = xef_ref[0]
    xev = xev_ref[0]
    xnc = xnc_ref[0]

    x8 = xef @ px_ref[...]
    oh8 = (x8 == i8_ref[...]).astype(jnp.float32)
    feats = oh8 + xev @ pv_ref[...]
    e = feats @ w8_ref[...]

    h = xnc @ wco2_ref[...] + ne2_ref[0][None, :]

    a_src = h @ ws2_ref[...]
    a_dst = h @ wd2_ref[...] + bm2_ref[0][None, :]
    pre = (e @ we2_ref[...]).reshape(N, N, H2)
    e3 = e.reshape(N, N, H2)
    pre = pre + a_src[:, None, :] + a_dst[None, :, :]
    m = jnp.maximum(pre, 0.0)
    gm = jax.nn.sigmoid(e3) * m
    agg = jnp.sum(gm, axis=0)

    h_new = jnp.maximum(h + agg @ wn2_ref[...] + bn2_ref[0][None, :], 0.0)

    t_src = h_new @ wcs2_ref[...]
    t_dst = h_new @ wcd2_ref[...]
    e_new = jnp.maximum(e3 + m, 0.0)
    y = (e_new.reshape(NN, H2) @ wce2_ref[...]).reshape(N, N, 4)
    y = y + t_src[:, None, :] + t_dst[None, :, :]
    out_ref[0] = y[:, :, 0:2]
    out_ref[1] = y[:, :, 2:4]


def _blockdiag(w):
    z = jnp.zeros_like(w)
    return jnp.concatenate([jnp.concatenate([w, z], axis=1),
                            jnp.concatenate([z, w], axis=1)], axis=0)


@jax.jit
def kernel(x_edges, x_edges_values, x_nodes, x_nodes_coord, edge_embed,
           node_embed, W_dist, W_coord, W_msg, b_msg, W_node, b_node,
           W_cls, b_cls, logit_noedge):
    B, N = x_nodes.shape
    H = node_embed.shape[1]
    C = W_cls.shape[1]
    P = B // 2

    pair = lambda x: jnp.stack([x[0::2], x[1::2]], axis=-1)
    xef = pair(x_edges.astype(jnp.float32)).reshape(P, N * N, 2)
    xev = pair(x_edges_values).reshape(P, N * N, 2)
    xnc = jnp.concatenate([x_nodes_coord[0::2], x_nodes_coord[1::2]],
                          axis=-1)

    f32 = jnp.float32
    px = jnp.array([[1, 1, 1, 0, 0, 0, 0, 0],
                    [0, 0, 0, 0, 1, 1, 1, 0]], dtype=f32)
    pv = jnp.array([[0, 0, 0, 1, 0, 0, 0, 0],
                    [0, 0, 0, 0, 0, 0, 0, 1]], dtype=f32)
    i8 = jnp.array([[0, 1, 2, 9, 0, 1, 2, 9]], dtype=f32)
    w8 = _blockdiag(jnp.concatenate([edge_embed, W_dist], axis=0))
    ne2 = jnp.concatenate([node_embed[0:1], node_embed[0:1]], axis=1)
    wco2 = _blockdiag(W_coord)
    we2 = _blockdiag(W_msg[:H])
    ws2 = _blockdiag(W_msg[H:2 * H])
    wd2 = _blockdiag(W_msg[2 * H:])
    bm2 = jnp.concatenate([b_msg, b_msg]).reshape(1, 2 * H)
    wn2 = _blockdiag(W_node)
    bn2 = jnp.concatenate([b_node, b_node]).reshape(1, 2 * H)
    wce2 = _blockdiag(W_cls[:H])
    wcs2 = _blockdiag(W_cls[H:2 * H])
    wcd2 = _blockdiag(W_cls[2 * H:]) + jnp.concatenate(
        [b_cls, b_cls]).reshape(1, 2 * C)

    full = lambda shape: pl.BlockSpec(shape, lambda p: (0,) * len(shape))
    out = pl.pallas_call(
        _gcn_fused_kernel,
        grid=(P,),
        in_specs=[
            pl.BlockSpec((1, N * N, 2), lambda p: (p, 0, 0)),
            pl.BlockSpec((1, N * N, 2), lambda p: (p, 0, 0)),
            pl.BlockSpec((1, N, 4), lambda p: (p, 0, 0)),
            full((2, 8)),
            full((2, 8)),
            full((1, 8)),
            full((8, 2 * H)),
            full((1, 2 * H)),
            full((4, 2 * H)),
            full((2 * H, 2 * H)),
            full((2 * H, 2 * H)),
            full((2 * H, 2 * H)),
            full((1, 2 * H)),
            full((2 * H, 2 * H)),
            full((1, 2 * H)),
            full((2 * H, 2 * C)),
            full((2 * H, 2 * C)),
            full((2 * H, 2 * C)),
        ],
        out_specs=pl.BlockSpec((2, N, N, C), lambda p: (p, 0, 0, 0)),
        out_shape=jax.ShapeDtypeStruct((B, N, N, C), jnp.float32),
    )(xef, xev, xnc, px, pv, i8, w8, ne2, wco2, we2, ws2, wd2, bm2, wn2,
      bn2, wce2, wcs2, wcd2)
    return out


# speedup vs baseline: 1.0459x; 1.0459x over previous
import jax
import jax.numpy as jnp
from jax.experimental import pallas as pl


def _gcn_fused_kernel(xe_ref, xev_ref, xnc_ref, ee2_ref, ne2_ref, wd2_ref,
                      wco2_ref, we2_ref, ws2_ref, wsd2_ref, bm2_ref,
                      wn2_ref, bn2_ref, wce2_ref, wcs2_ref, wcd2_ref,
                      out_ref):
    N = xe_ref.shape[1]
    H2 = ne2_ref.shape[1]
    H = H2 // 2
    f32 = jnp.float32

    d0 = ee2_ref[0] - ee2_ref[2]
    d1 = ee2_ref[1] - ee2_ref[2]
    half = jnp.concatenate(
        [jnp.zeros((1, H), f32), jnp.ones((1, H), f32)], axis=1)

    def bcast_pair(a0, a1):
        return a0[:, :, None] + (a1 - a0)[:, :, None] * half[0][None, None, :]

    xe0 = xe_ref[0].astype(f32)
    xe1 = xe_ref[1].astype(f32)
    c0 = bcast_pair((xe0 == 0.0).astype(f32), (xe1 == 0.0).astype(f32))
    c1 = bcast_pair((xe0 == 1.0).astype(f32), (xe1 == 1.0).astype(f32))
    xv = bcast_pair(xev_ref[0], xev_ref[1])
    e3 = (ee2_ref[2][None, None, :] + c0 * d0[None, None, :]
          + c1 * d1[None, None, :] + xv * wd2_ref[0][None, None, :])

    xnc = jnp.concatenate([xnc_ref[0], xnc_ref[1]], axis=1)
    h = xnc @ wco2_ref[...] + ne2_ref[0][None, :]

    a_src = h @ ws2_ref[...]
    a_dst = h @ wsd2_ref[...] + bm2_ref[0][None, :]
    e = e3.reshape(N * N, H2)
    pre = (e @ we2_ref[...]).reshape(N, N, H2)
    pre = pre + a_src[:, None, :] + a_dst[None, :, :]
    m = jnp.maximum(pre, 0.0)
    gm = jax.nn.sigmoid(e3) * m
    agg = jnp.sum(gm, axis=0)

    h_new = jnp.maximum(h + agg @ wn2_ref[...] + bn2_ref[0][None, :], 0.0)

    t_src = h_new @ wcs2_ref[...]
    t_dst = h_new @ wcd2_ref[...]
    e_new = jnp.maximum(e3 + m, 0.0)
    y = (e_new.reshape(N * N, H2) @ wce2_ref[...]).reshape(N, N, 4)
    y = y + t_src[:, None, :] + t_dst[None, :, :]
    out_ref[0] = y[:, :, 0:2]
    out_ref[1] = y[:, :, 2:4]


def _blockdiag(w):
    z = jnp.zeros_like(w)
    return jnp.concatenate([jnp.concatenate([w, z], axis=1),
                            jnp.concatenate([z, w], axis=1)], axis=0)


@jax.jit
def kernel(x_edges, x_edges_values, x_nodes, x_nodes_coord, edge_embed,
           node_embed, W_dist, W_coord, W_msg, b_msg, W_node, b_node,
           W_cls, b_cls, logit_noedge):
    B, N = x_nodes.shape
    H = node_embed.shape[1]
    C = W_cls.shape[1]
    P = B // 2

    tile2 = lambda v: jnp.concatenate([v, v], axis=-1)
    ee2 = tile2(edge_embed)
    ne2 = tile2(node_embed[0:1])
    wd2 = tile2(W_dist)
    wco2 = _blockdiag(W_coord)
    we2 = _blockdiag(W_msg[:H])
    ws2 = _blockdiag(W_msg[H:2 * H])
    wsd2 = _blockdiag(W_msg[2 * H:])
    bm2 = tile2(b_msg.reshape(1, H))
    wn2 = _blockdiag(W_node)
    bn2 = tile2(b_node.reshape(1, H))
    wce2 = _blockdiag(W_cls[:H])
    wcs2 = _blockdiag(W_cls[H:2 * H])
    wcd2 = _blockdiag(W_cls[2 * H:]) + tile2(b_cls.reshape(1, C))

    full = lambda shape: pl.BlockSpec(shape, lambda p: (0,) * len(shape))
    out = pl.pallas_call(
        _gcn_fused_kernel,
        grid=(P,),
        in_specs=[
            pl.BlockSpec((2, N, N), lambda p: (p, 0, 0)),
            pl.BlockSpec((2, N, N), lambda p: (p, 0, 0)),
            pl.BlockSpec((2, N, 2), lambda p: (p, 0, 0)),
            full((3, 2 * H)),
            full((1, 2 * H)),
            full((1, 2 * H)),
            full((4, 2 * H)),
            full((2 * H, 2 * H)),
            full((2 * H, 2 * H)),
            full((2 * H, 2 * H)),
            full((1, 2 * H)),
            full((2 * H, 2 * H)),
            full((1, 2 * H)),
            full((2 * H, 2 * C)),
            full((2 * H, 2 * C)),
            full((2 * H, 2 * C)),
        ],
        out_specs=pl.BlockSpec((2, N, N, C), lambda p: (p, 0, 0, 0)),
        out_shape=jax.ShapeDtypeStruct((B, N, N, C), jnp.float32),
    )(x_edges, x_edges_values, x_nodes_coord, ee2, ne2, wd2, wco2, we2,
      ws2, wsd2, bm2, wn2, bn2, wce2, wcs2, wcd2)
    return out


# speedup vs baseline: 1.1926x; 1.1402x over previous
import jax
import jax.numpy as jnp
from jax.experimental import pallas as pl


def _gcn_fused_kernel(xe_ref, xev_ref, xnc_ref, ee2_ref, ne2_ref, wd2_ref,
                      wco2_ref, we2_ref, ws2_ref, wsd2_ref, bm2_ref,
                      wn2_ref, bn2_ref, wce2_ref, wcs2_ref, wcd2_ref,
                      out_ref):
    N = xe_ref.shape[1]
    H2 = ne2_ref.shape[1]
    H = H2 // 2
    f32 = jnp.float32

    d0 = ee2_ref[0] - ee2_ref[2]
    d1 = ee2_ref[1] - ee2_ref[2]
    half = jnp.concatenate(
        [jnp.zeros((1, H), f32), jnp.ones((1, H), f32)], axis=1)

    def bcast_pair(a0, a1):
        return a0[:, :, None] + (a1 - a0)[:, :, None] * half[0][None, None, :]

    xe0 = xe_ref[0].astype(f32)
    xe1 = xe_ref[1].astype(f32)
    c0 = bcast_pair((xe0 == 0.0).astype(f32), (xe1 == 0.0).astype(f32))
    c1 = bcast_pair((xe0 == 1.0).astype(f32), (xe1 == 1.0).astype(f32))
    xv = bcast_pair(xev_ref[0], xev_ref[1])
    e3 = (ee2_ref[2][None, None, :] + c0 * d0[None, None, :]
          + c1 * d1[None, None, :] + xv * wd2_ref[0][None, None, :])

    xnc = jnp.concatenate([xnc_ref[0], xnc_ref[1]], axis=1)
    h = xnc @ wco2_ref[...] + ne2_ref[0][None, :]

    a_src = h @ ws2_ref[...]
    a_dst = h @ wsd2_ref[...] + bm2_ref[0][None, :]
    e = e3.reshape(N * N, H2)
    pre = (e @ we2_ref[...]).reshape(N, N, H2)
    pre = pre + a_src[:, None, :] + a_dst[None, :, :]
    m = jnp.maximum(pre, 0.0)
    gm = jax.nn.sigmoid(e3) * m
    agg = jnp.sum(gm, axis=0)

    h_new = jnp.maximum(h + agg @ wn2_ref[...] + bn2_ref[0][None, :], 0.0)

    t_src = h_new @ wcs2_ref[...]
    t_dst = h_new @ wcd2_ref[...]
    e_new = jnp.maximum(e3 + m, 0.0)
    y = (e_new.reshape(N * N, H2) @ wce2_ref[...]).reshape(N, N, 4)
    y = y + t_src[:, None, :] + t_dst[None, :, :]
    out_ref[0] = y[:, :, 0:2].reshape(N, 2 * N)
    out_ref[1] = y[:, :, 2:4].reshape(N, 2 * N)


def _blockdiag(w):
    z = jnp.zeros_like(w)
    return jnp.concatenate([jnp.concatenate([w, z], axis=1),
                            jnp.concatenate([z, w], axis=1)], axis=0)


@jax.jit
def kernel(x_edges, x_edges_values, x_nodes, x_nodes_coord, edge_embed,
           node_embed, W_dist, W_coord, W_msg, b_msg, W_node, b_node,
           W_cls, b_cls, logit_noedge):
    B, N = x_nodes.shape
    H = node_embed.shape[1]
    C = W_cls.shape[1]
    P = B // 2

    tile2 = lambda v: jnp.concatenate([v, v], axis=-1)
    ee2 = tile2(edge_embed)
    ne2 = tile2(node_embed[0:1])
    wd2 = tile2(W_dist)
    wco2 = _blockdiag(W_coord)
    we2 = _blockdiag(W_msg[:H])
    ws2 = _blockdiag(W_msg[H:2 * H])
    wsd2 = _blockdiag(W_msg[2 * H:])
    bm2 = tile2(b_msg.reshape(1, H))
    wn2 = _blockdiag(W_node)
    bn2 = tile2(b_node.reshape(1, H))
    wce2 = _blockdiag(W_cls[:H])
    wcs2 = _blockdiag(W_cls[H:2 * H])
    wcd2 = _blockdiag(W_cls[2 * H:]) + tile2(b_cls.reshape(1, C))

    full = lambda shape: pl.BlockSpec(shape, lambda p: (0,) * len(shape))
    out = pl.pallas_call(
        _gcn_fused_kernel,
        grid=(P,),
        in_specs=[
            pl.BlockSpec((2, N, N), lambda p: (p, 0, 0)),
            pl.BlockSpec((2, N, N), lambda p: (p, 0, 0)),
            pl.BlockSpec((2, N, 2), lambda p: (p, 0, 0)),
            full((3, 2 * H)),
            full((1, 2 * H)),
            full((1, 2 * H)),
            full((4, 2 * H)),
            full((2 * H, 2 * H)),
            full((2 * H, 2 * H)),
            full((2 * H, 2 * H)),
            full((1, 2 * H)),
            full((2 * H, 2 * H)),
            full((1, 2 * H)),
            full((2 * H, 2 * C)),
            full((2 * H, 2 * C)),
            full((2 * H, 2 * C)),
        ],
        out_specs=pl.BlockSpec((2, N, N * C), lambda p: (p, 0, 0)),
        out_shape=jax.ShapeDtypeStruct((B, N, N * C), jnp.float32),
    )(x_edges, x_edges_values, x_nodes_coord, ee2, ne2, wd2, wco2, we2,
      ws2, wsd2, bm2, wn2, bn2, wce2, wcs2, wcd2)
    return out.reshape(B, N, N, C)


# speedup vs baseline: 1.8802x; 1.5766x over previous
import jax
import jax.numpy as jnp
from jax.experimental import pallas as pl


def _gcn_fused_kernel(xe_ref, xev_ref, xnc_ref, w8_ref, ne2_ref,
                      wco2_ref, we2_ref, ws2_ref, wsd2_ref, bm2_ref,
                      wn2_ref, bn2_ref, wce2_ref, wcs2_ref, wcd2_ref,
                      out_ref):
    N = xe_ref.shape[1]
    H2 = ne2_ref.shape[1]
    f32 = jnp.float32

    xe0 = xe_ref[0]
    xe1 = xe_ref[1]
    feats = jnp.stack(
        [(xe0 == 0).astype(f32), (xe0 == 1).astype(f32),
         (xe0 == 2).astype(f32), xev_ref[0],
         (xe1 == 0).astype(f32), (xe1 == 1).astype(f32),
         (xe1 == 2).astype(f32), xev_ref[1]], axis=1)
    e3 = jax.lax.dot_general(
        feats, w8_ref[...],
        dimension_numbers=(((1,), (0,)), ((), ())))

    xnc = jnp.concatenate([xnc_ref[0], xnc_ref[1]], axis=1)
    h = xnc @ wco2_ref[...] + ne2_ref[0][None, :]

    a_src = h @ ws2_ref[...]
    a_dst = h @ wsd2_ref[...] + bm2_ref[0][None, :]
    e = e3.reshape(N * N, H2)
    pre = (e @ we2_ref[...]).reshape(N, N, H2)
    pre = pre + a_src[:, None, :] + a_dst[None, :, :]
    m = jnp.maximum(pre, 0.0)
    gm = jax.nn.sigmoid(e3) * m
    agg = jnp.sum(gm, axis=0)

    h_new = jnp.maximum(h + agg @ wn2_ref[...] + bn2_ref[0][None, :], 0.0)

    t_src = h_new @ wcs2_ref[...]
    t_dst = h_new @ wcd2_ref[...]
    e_new = jnp.maximum(e3 + m, 0.0)
    y = (e_new.reshape(N * N, H2) @ wce2_ref[...]).reshape(N, N, 4)
    y = y + t_src[:, None, :] + t_dst[None, :, :]
    out_ref[0] = y[:, :, 0:2].reshape(N, 2 * N)
    out_ref[1] = y[:, :, 2:4].reshape(N, 2 * N)


def _blockdiag(w):
    z = jnp.zeros_like(w)
    return jnp.concatenate([jnp.concatenate([w, z], axis=1),
                            jnp.concatenate([z, w], axis=1)], axis=0)


@jax.jit
def kernel(x_edges, x_edges_values, x_nodes, x_nodes_coord, edge_embed,
           node_embed, W_dist, W_coord, W_msg, b_msg, W_node, b_node,
           W_cls, b_cls, logit_noedge):
    B, N = x_nodes.shape
    H = node_embed.shape[1]
    C = W_cls.shape[1]
    P = B // 2

    tile2 = lambda v: jnp.concatenate([v, v], axis=-1)
    ne2 = tile2(node_embed[0:1])
    w8 = _blockdiag(jnp.concatenate([edge_embed, W_dist], axis=0))
    wco2 = _blockdiag(W_coord)
    we2 = _blockdiag(W_msg[:H])
    ws2 = _blockdiag(W_msg[H:2 * H])
    wsd2 = _blockdiag(W_msg[2 * H:])
    bm2 = tile2(b_msg.reshape(1, H))
    wn2 = _blockdiag(W_node)
    bn2 = tile2(b_node.reshape(1, H))
    wce2 = _blockdiag(W_cls[:H])
    wcs2 = _blockdiag(W_cls[H:2 * H])
    wcd2 = _blockdiag(W_cls[2 * H:]) + tile2(b_cls.reshape(1, C))

    full = lambda shape: pl.BlockSpec(shape, lambda p: (0,) * len(shape))
    out = pl.pallas_call(
        _gcn_fused_kernel,
        grid=(P,),
        in_specs=[
            pl.BlockSpec((2, N, N), lambda p: (p, 0, 0)),
            pl.BlockSpec((2, N, N), lambda p: (p, 0, 0)),
            pl.BlockSpec((2, N, 2), lambda p: (p, 0, 0)),
            full((8, 2 * H)),
            full((1, 2 * H)),
            full((4, 2 * H)),
            full((2 * H, 2 * H)),
            full((2 * H, 2 * H)),
            full((2 * H, 2 * H)),
            full((1, 2 * H)),
            full((2 * H, 2 * H)),
            full((1, 2 * H)),
            full((2 * H, 2 * C)),
            full((2 * H, 2 * C)),
            full((2 * H, 2 * C)),
        ],
        out_specs=pl.BlockSpec((2, N, N * C), lambda p: (p, 0, 0)),
        out_shape=jax.ShapeDtypeStruct((B, N, N * C), jnp.float32),
    )(x_edges, x_edges_values, x_nodes_coord, w8, ne2, wco2, we2,
      ws2, wsd2, bm2, wn2, bn2, wce2, wcs2, wcd2)
    return out.reshape(B, N, N, C)
